# baseline (speedup 1.0000x reference)
import jax
import jax.numpy as jnp
from jax.experimental import pallas as pl
from jax.experimental.pallas import tpu as pltpu

_F, _N, _B, _C = 2, 2, 16384, 1000
_M = 0.3
_T = 0.1
_LAMBDA_P = 0.1
_RHO = 0.01

_BM = 512
_NB = _B // _BM
_NS = _F * _N


def _loss_body(y_ref, lab_ref, pc_ref, out_ref):
    s = pl.program_id(0)
    j = pl.program_id(1)

    @pl.when((s == 0) & (j == 0))
    def _init():
        out_ref[0, 0] = 0.0

    yb = y_ref[0, 0]
    lab = lab_ref[...]
    cols = jax.lax.broadcasted_iota(jnp.int32, (_BM, _C), 1)
    mask = cols == lab
    fy = jnp.sum(jnp.where(mask, yb, 0.0), axis=1, keepdims=True)
    fnym = jnp.max(jnp.where(mask, -1e10, yb), axis=1, keepdims=True)
    l = jnp.maximum(_M + _T - fy, 0.0) + jnp.maximum(_M + fnym, 0.0)
    out_ref[0, 0] += jnp.sum(l) * (1.0 / (_NS * _B))

    @pl.when((s == _NS - 1) & (j == _NB - 1))
    def _fini():
        pc = pc_ref[0, 0]
        out_ref[0, 0] += _LAMBDA_P * pc + (_RHO / 2.0) * pc * pc


def kernel(y, label, power_ratio, power_consumption):
    del power_ratio
    lab2 = label[:, None]
    pc = power_consumption.reshape(1, 1)

    out = pl.pallas_call(
        _loss_body,
        grid=(_NS, _NB),
        in_specs=[
            pl.BlockSpec((1, 1, _BM, _C), lambda s, j: (s // _N, s % _N, j, 0)),
            pl.BlockSpec((_BM, 1), lambda s, j: (j, 0)),
            pl.BlockSpec(memory_space=pltpu.SMEM),
        ],
        out_specs=pl.BlockSpec(memory_space=pltpu.SMEM),
        out_shape=jax.ShapeDtypeStruct((1, 1), jnp.float32),
        compiler_params=pltpu.CompilerParams(
            dimension_semantics=("arbitrary", "arbitrary"),
        ),
    )(y, lab2, pc)
    return out.reshape(1)


# speedup vs baseline: 1.1880x; 1.1880x over previous
import jax
import jax.numpy as jnp
from jax.experimental import pallas as pl
from jax.experimental.pallas import tpu as pltpu

_F, _N, _B, _C = 2, 2, 16384, 1000
_M = 0.3
_T = 0.1
_LAMBDA_P = 0.1
_RHO = 0.01

_BM = 512
_NSTREAM = 4
_ROWS_PER_STEP = _BM * _NSTREAM
_NB = _B // _ROWS_PER_STEP
_NS = _F * _N


def _loss_body(*refs):
    y_refs = refs[:_NSTREAM]
    lab_ref, pc_ref, out_ref = refs[_NSTREAM:]
    s = pl.program_id(0)
    j = pl.program_id(1)

    @pl.when((s == 0) & (j == 0))
    def _init():
        out_ref[0, 0] = 0.0

    cols = jax.lax.broadcasted_iota(jnp.int32, (_BM, _C), 1)
    acc = 0.0
    for k in range(_NSTREAM):
        yb = y_refs[k][0, 0]
        lab = lab_ref[pl.ds(k * _BM, _BM), :]
        mask = cols == lab
        fy = jnp.sum(jnp.where(mask, yb, 0.0), axis=1, keepdims=True)
        fnym = jnp.max(jnp.where(mask, -1e10, yb), axis=1, keepdims=True)
        l = jnp.maximum(_M + _T - fy, 0.0) + jnp.maximum(_M + fnym, 0.0)
        acc += jnp.sum(l)
    out_ref[0, 0] += acc * (1.0 / (_NS * _B))

    @pl.when((s == _NS - 1) & (j == _NB - 1))
    def _fini():
        pc = pc_ref[0, 0]
        out_ref[0, 0] += _LAMBDA_P * pc + (_RHO / 2.0) * pc * pc


def _y_spec(k):
    return pl.BlockSpec(
        (1, 1, _BM, _C),
        lambda s, j, k=k: (s // _N, s % _N, j * _NSTREAM + k, 0),
    )


def kernel(y, label, power_ratio, power_consumption):
    del power_ratio
    lab2 = label[:, None]
    pc = power_consumption.reshape(1, 1)

    out = pl.pallas_call(
        _loss_body,
        grid=(_NS, _NB),
        in_specs=(
            [_y_spec(k) for k in range(_NSTREAM)]
            + [
                pl.BlockSpec((_ROWS_PER_STEP, 1), lambda s, j: (j, 0)),
                pl.BlockSpec(memory_space=pltpu.SMEM),
            ]
        ),
        out_specs=pl.BlockSpec(memory_space=pltpu.SMEM),
        out_shape=jax.ShapeDtypeStruct((1, 1), jnp.float32),
        compiler_params=pltpu.CompilerParams(
            dimension_semantics=("arbitrary", "arbitrary"),
        ),
    )(*([y] * _NSTREAM), lab2, pc)
    return out.reshape(1)


# speedup vs baseline: 1.2094x; 1.0180x over previous
import jax
import jax.numpy as jnp
from jax.experimental import pallas as pl
from jax.experimental.pallas import tpu as pltpu

_F, _N, _B, _C = 2, 2, 16384, 1000
_M = 0.3
_T = 0.1
_LAMBDA_P = 0.1
_RHO = 0.01

_BM = 1024
_NSTREAM = 4
_ROWS_PER_STEP = _BM * _NSTREAM
_NB = _B // _ROWS_PER_STEP
_NS = _F * _N


def _loss_body(*refs):
    y_refs = refs[:_NSTREAM]
    lab_ref, pc_ref, out_ref = refs[_NSTREAM:]
    s = pl.program_id(0)
    j = pl.program_id(1)

    @pl.when((s == 0) & (j == 0))
    def _init():
        out_ref[0, 0] = 0.0

    cols = jax.lax.broadcasted_iota(jnp.int32, (_BM, _C), 1)
    acc = 0.0
    for k in range(_NSTREAM):
        yb = y_refs[k][0, 0]
        lab = lab_ref[pl.ds(k * _BM, _BM), :]
        mask = cols == lab
        fy = jnp.sum(jnp.where(mask, yb, 0.0), axis=1, keepdims=True)
        fnym = jnp.max(jnp.where(mask, -1e10, yb), axis=1, keepdims=True)
        l = jnp.maximum(_M + _T - fy, 0.0) + jnp.maximum(_M + fnym, 0.0)
        acc += jnp.sum(l)
    out_ref[0, 0] += acc * (1.0 / (_NS * _B))

    @pl.when((s == _NS - 1) & (j == _NB - 1))
    def _fini():
        pc = pc_ref[0, 0]
        out_ref[0, 0] += _LAMBDA_P * pc + (_RHO / 2.0) * pc * pc


def _y_spec(k):
    return pl.BlockSpec(
        (1, 1, _BM, _C),
        lambda s, j, k=k: (s // _N, s % _N, j * _NSTREAM + k, 0),
    )


def kernel(y, label, power_ratio, power_consumption):
    del power_ratio
    lab2 = label[:, None]
    pc = power_consumption.reshape(1, 1)

    out = pl.pallas_call(
        _loss_body,
        grid=(_NS, _NB),
        in_specs=(
            [_y_spec(k) for k in range(_NSTREAM)]
            + [
                pl.BlockSpec((_ROWS_PER_STEP, 1), lambda s, j: (j, 0)),
                pl.BlockSpec(memory_space=pltpu.SMEM),
            ]
        ),
        out_specs=pl.BlockSpec(memory_space=pltpu.SMEM),
        out_shape=jax.ShapeDtypeStruct((1, 1), jnp.float32),
        compiler_params=pltpu.CompilerParams(
            dimension_semantics=("arbitrary", "arbitrary"),
        ),
    )(*([y] * _NSTREAM), lab2, pc)
    return out.reshape(1)
